# baseline (speedup 1.0000x reference)
import functools

import jax
import jax.numpy as jnp
from jax import lax
from jax.experimental import pallas as pl
from jax.experimental.pallas import tpu as pltpu
from jax.experimental.pallas import tpu_sc as plsc

F = 8
HP = 2056
HPW = HP // 2
NTAB = HP * HPW
NB = 4
W = 512
H = 512
NW = 32
ROWS_PW = H // NW


def _sc_body(uv_ref, tab_ref, out_ref, u_v, v_v, wx_v, wy_v, par_v, idx_v,
             rt_v, rts_v, out_v, sem):
    wid = lax.axis_index("s") * 2 + lax.axis_index("c")
    iota16 = lax.iota(jnp.int32, 16)
    row0 = wid * ROWS_PW

    def do_row(k):
        h = row0 + k

        for qb in range(NB):
            pltpu.sync_copy(uv_ref.at[qb, 0, h], u_v.at[qb])
            pltpu.sync_copy(uv_ref.at[qb, 1, h], v_v.at[qb])
            for s in range(4):
                def phase_a(j, carry, qb=qb, s=s):
                    off = (s * 8 + j) * 16
                    yf = u_v[qb, pl.ds(off, 16)] * 2048.0 + 4.0
                    yf = jnp.minimum(jnp.maximum(yf, 0.0), float(HP - 1))
                    yi = jnp.minimum(yf.astype(jnp.int32), HP - 2)
                    xf = v_v[qb, pl.ds(off, 16)] * 2048.0 + 4.0
                    xf = jnp.minimum(jnp.maximum(xf, 0.0), float(HP - 1))
                    xi = jnp.minimum(xf.astype(jnp.int32), HP - 2)
                    wy_v[qb, pl.ds(off, 16)] = yf - yi.astype(jnp.float32)
                    wx_v[qb, pl.ds(off, 16)] = xf - xi.astype(jnp.float32)
                    par_v[qb, pl.ds(off, 16)] = jnp.bitwise_and(xi, 1)
                    i0 = yi * HPW + lax.shift_right_logical(xi, 1)
                    idx_v[qb, 0, s, pl.ds(j * 16, 16)] = i0
                    idx_v[qb, 1, s, pl.ds(j * 16, 16)] = i0 + 1
                    idx_v[qb, 2, s, pl.ds(j * 16, 16)] = i0 + HPW
                    idx_v[qb, 3, s, pl.ds(j * 16, 16)] = jnp.minimum(
                        i0 + HPW + 1, NTAB - 1)
                    return carry
                lax.fori_loop(0, 8, phase_a, None)

        for qb in range(NB):
            descs = []
            for cr in range(4):
                for s in range(4):
                    descs.append(pltpu.async_copy(
                        tab_ref.at[idx_v.at[qb, cr, s]],
                        rt_v.at[pl.ds(cr * W + s * 128, 128)], sem))
            for d in descs:
                d.wait()

            def stage(q, carry):
                base = q * 64
                rts_v[pl.ds(base, 16)] = rt_v[q]
                rts_v[pl.ds(base + 16, 16)] = rt_v[W + q]
                rts_v[pl.ds(base + 32, 16)] = rt_v[2 * W + q]
                rts_v[pl.ds(base + 48, 16)] = rt_v[3 * W + q]
                return carry
            lax.fori_loop(0, W, stage, None)

            def combine(g, carry, qb=qb):
                off = g * 16
                pav = par_v[qb, pl.ds(off, 16)] * 8
                qa = (off + iota16) * 64 + pav
                for wb in range(NB):
                    wx = wx_v[wb, pl.ds(off, 16)]
                    wy = wy_v[wb, pl.ds(off, 16)]
                    for ci in range(2):
                        ch = 2 * wb + ci
                        a = plsc.load_gather(rts_v, [qa + ch])
                        b_ = plsc.load_gather(rts_v, [qa + (ch + 8)])
                        cc = plsc.load_gather(rts_v, [qa + (ch + 32)])
                        dd = plsc.load_gather(rts_v, [qa + (ch + 40)])
                        top = a + wx * (b_ - a)
                        bot = cc + wx * (dd - cc)
                        out_v[ch, pl.ds(off, 16)] = top + wy * (bot - top)
                return carry
            lax.fori_loop(0, W // 16, combine, None)

            for ch in range(F):
                oi = ch * NB + qb
                pltpu.sync_copy(out_v.at[ch],
                                out_ref.at[oi // F, oi % F, h])

    def row_loop(k, carry):
        do_row(k)
        return carry
    lax.fori_loop(0, ROWS_PW, row_loop, None)


@jax.jit
def kernel(uv, feature_img):
    tab = jnp.transpose(feature_img, (1, 2, 0)).reshape(NTAB, 2 * F)
    run = functools.partial(
        pl.kernel,
        out_type=jax.ShapeDtypeStruct((NB, F, H, W), jnp.float32),
        mesh=plsc.VectorSubcoreMesh(core_axis_name="c", subcore_axis_name="s"),
        compiler_params=pltpu.CompilerParams(
            needs_layout_passes=False, use_tc_tiling_on_sc=False),
        scratch_types=[
            pltpu.VMEM((NB, W), jnp.float32),
            pltpu.VMEM((NB, W), jnp.float32),
            pltpu.VMEM((NB, W), jnp.float32),
            pltpu.VMEM((NB, W), jnp.float32),
            pltpu.VMEM((NB, W), jnp.int32),
            pltpu.VMEM((NB, 4, 4, 128), jnp.int32),
            pltpu.VMEM((4 * W, 2 * F), jnp.float32),
            pltpu.VMEM((4 * W * 2 * F,), jnp.float32),
            pltpu.VMEM((F, W), jnp.float32),
            pltpu.SemaphoreType.DMA,
        ],
    )(_sc_body)
    return run(uv, tab)


# speedup vs baseline: 1.0001x; 1.0001x over previous
import functools

import jax
import jax.numpy as jnp
from jax import lax
from jax.experimental import pallas as pl
from jax.experimental.pallas import tpu as pltpu
from jax.experimental.pallas import tpu_sc as plsc

F = 8
HP = 2056
HPW = HP // 2
NTAB = HP * HPW
NB = 4
W = 512
H = 512
NW = 32
ROWS_PW = H // NW


def _sc_body(uv_ref, tab_ref, out_ref, u_v, v_v, wx_v, wy_v, par_v, idx_v,
             rt_v, rts_v, out_v, sem):
    wid = lax.axis_index("s") * 2 + lax.axis_index("c")
    iota16 = lax.iota(jnp.int32, 16)
    row0 = wid * ROWS_PW

    def do_row(k):
        h = row0 + k

        for qb in range(NB):
            pltpu.sync_copy(uv_ref.at[qb, 0, h], u_v.at[qb])
            pltpu.sync_copy(uv_ref.at[qb, 1, h], v_v.at[qb])
            for s in range(4):
                def phase_a(j, carry, qb=qb, s=s):
                    off = (s * 8 + j) * 16
                    yf = u_v[qb, pl.ds(off, 16)] * 2048.0 + 4.0
                    yf = jnp.minimum(jnp.maximum(yf, 0.0), float(HP - 1))
                    yi = jnp.minimum(yf.astype(jnp.int32), HP - 2)
                    xf = v_v[qb, pl.ds(off, 16)] * 2048.0 + 4.0
                    xf = jnp.minimum(jnp.maximum(xf, 0.0), float(HP - 1))
                    xi = jnp.minimum(xf.astype(jnp.int32), HP - 2)
                    wy_v[qb, pl.ds(off, 16)] = yf - yi.astype(jnp.float32)
                    wx_v[qb, pl.ds(off, 16)] = xf - xi.astype(jnp.float32)
                    par_v[qb, pl.ds(off, 16)] = jnp.bitwise_and(xi, 1)
                    i0 = yi * HPW + lax.shift_right_logical(xi, 1)
                    idx_v[qb, 0, s, pl.ds(j * 16, 16)] = i0
                    idx_v[qb, 1, s, pl.ds(j * 16, 16)] = i0 + 1
                    idx_v[qb, 2, s, pl.ds(j * 16, 16)] = i0 + HPW
                    idx_v[qb, 3, s, pl.ds(j * 16, 16)] = jnp.minimum(
                        i0 + HPW + 1, NTAB - 1)
                    return carry
                lax.fori_loop(0, 8, phase_a, None)

        for qb in range(NB):
            descs = []
            for cr in range(4):
                for s in range(4):
                    descs.append(pltpu.async_copy(
                        tab_ref.at[idx_v.at[qb, cr, s]],
                        rt_v.at[pl.ds(cr * W + s * 128, 128)], sem))
            for d in descs:
                d.wait()

            def stage(q, carry):
                base = q * 64
                rts_v[pl.ds(base, 16)] = rt_v[q]
                rts_v[pl.ds(base + 16, 16)] = rt_v[W + q]
                rts_v[pl.ds(base + 32, 16)] = rt_v[2 * W + q]
                rts_v[pl.ds(base + 48, 16)] = rt_v[3 * W + q]
                return carry
            lax.fori_loop(0, W, stage, None)

            def combine(g, carry, qb=qb):
                off = g * 16
                pav = par_v[qb, pl.ds(off, 16)] * 8
                qa = (off + iota16) * 64 + pav
                for wb in range(NB):
                    wx = wx_v[wb, pl.ds(off, 16)]
                    wy = wy_v[wb, pl.ds(off, 16)]
                    for ci in range(2):
                        ch = 2 * wb + ci
                        a = plsc.load_gather(rts_v, [qa + ch])
                        b_ = plsc.load_gather(rts_v, [qa + (ch + 8)])
                        cc = plsc.load_gather(rts_v, [qa + (ch + 32)])
                        dd = plsc.load_gather(rts_v, [qa + (ch + 40)])
                        top = a + wx * (b_ - a)
                        bot = cc + wx * (dd - cc)
                        out_v[ch, pl.ds(off, 16)] = top + wy * (bot - top)
                return carry
            lax.fori_loop(0, W // 16, combine, None)

            for ch in range(F):
                oi = ch * NB + qb
                pltpu.sync_copy(out_v.at[ch],
                                out_ref.at[oi // F, oi % F, h])

    def row_loop(k, carry):
        do_row(k)
        return carry
    lax.fori_loop(0, ROWS_PW, row_loop, None)


@jax.jit
def kernel(uv, feature_img):
    tab = lax.reshape(feature_img, (NTAB, 2 * F), dimensions=(1, 2, 0))
    run = functools.partial(
        pl.kernel,
        out_type=jax.ShapeDtypeStruct((NB, F, H, W), jnp.float32),
        mesh=plsc.VectorSubcoreMesh(core_axis_name="c", subcore_axis_name="s"),
        compiler_params=pltpu.CompilerParams(
            needs_layout_passes=False, use_tc_tiling_on_sc=False),
        scratch_types=[
            pltpu.VMEM((NB, W), jnp.float32),
            pltpu.VMEM((NB, W), jnp.float32),
            pltpu.VMEM((NB, W), jnp.float32),
            pltpu.VMEM((NB, W), jnp.float32),
            pltpu.VMEM((NB, W), jnp.int32),
            pltpu.VMEM((NB, 4, 4, 128), jnp.int32),
            pltpu.VMEM((4 * W, 2 * F), jnp.float32),
            pltpu.VMEM((4 * W * 2 * F,), jnp.float32),
            pltpu.VMEM((F, W), jnp.float32),
            pltpu.SemaphoreType.DMA,
        ],
    )(_sc_body)
    return run(uv, tab)


# speedup vs baseline: 1.0407x; 1.0405x over previous
import functools

import jax
import jax.numpy as jnp
from jax import lax
from jax.experimental import pallas as pl
from jax.experimental.pallas import tpu as pltpu
from jax.experimental.pallas import tpu_sc as plsc

F = 8
HP = 2056
HPW = HP // 2
NTAB = HP * HPW
NB = 4
W = 512
H = 512
NW = 32
ROWS_PW = H // NW
ROWS_A = (HP + NW - 1) // NW


def _review_body(t3_ref, tab_ref, bufa_v, bufb_v, lsem, ssem):
    wid = lax.axis_index("s") * 2 + lax.axis_index("c")
    y0 = wid * ROWS_A

    def wait_load(p):
        pltpu.make_async_copy(t3_ref.at[0], bufa_v.at[p], lsem).wait()

    def wait_store():
        pltpu.make_async_copy(
            bufb_v.at[0], tab_ref.at[pl.ds(0, HPW)], ssem).wait()

    @pl.when(y0 < HP)
    def _():
        pltpu.async_copy(t3_ref.at[y0], bufa_v.at[0], lsem)

    for i in range(ROWS_A):
        p = i % 2
        y = y0 + i

        @pl.when(y < HP)
        def _(p=p, y=y, i=i):
            wait_load(p)
            if i + 1 < ROWS_A:
                @pl.when(y + 1 < HP)
                def _():
                    pltpu.async_copy(t3_ref.at[y + 1], bufa_v.at[1 - p], lsem)
            if i >= 1:
                wait_store()

            def reshuffle(m, carry, p=p):
                bufb_v[p, m, :] = bufa_v[p, pl.ds(m * 16, 16)]
                return carry
            lax.fori_loop(0, HPW, reshuffle, None)
            pltpu.async_copy(bufb_v.at[p], tab_ref.at[pl.ds(y * HPW, HPW)],
                             ssem)

    for i in range(ROWS_A):
        @pl.when((y0 + i < HP) & (y0 + i + 1 >= HP))
        def _(i=i):
            wait_store()


def _sc_body(uv_ref, tab_ref, out_ref, u_v, v_v, wx_v, wy_v, par_v, idx_v,
             rt_v, rts_v, out_v, sem0, sem1):
    wid = lax.axis_index("s") * 2 + lax.axis_index("c")
    iota16 = lax.iota(jnp.int32, 16)
    row0 = wid * ROWS_PW

    def fire(qb, p):
        descs = []
        for cr in range(4):
            for s in range(4):
                descs.append(pltpu.async_copy(
                    tab_ref.at[idx_v.at[qb, cr, s]],
                    rt_v.at[p, pl.ds(cr * W + s * 128, 128)],
                    sem0 if p == 0 else sem1))
        return descs

    def do_row(k):
        h = row0 + k

        for qb in range(NB):
            pltpu.sync_copy(uv_ref.at[qb, 0, h], u_v.at[qb])
            pltpu.sync_copy(uv_ref.at[qb, 1, h], v_v.at[qb])
            for s in range(4):
                def phase_a(j, carry, qb=qb, s=s):
                    off = (s * 8 + j) * 16
                    yf = u_v[qb, pl.ds(off, 16)] * 2048.0 + 4.0
                    yf = jnp.minimum(jnp.maximum(yf, 0.0), float(HP - 1))
                    yi = jnp.minimum(yf.astype(jnp.int32), HP - 2)
                    xf = v_v[qb, pl.ds(off, 16)] * 2048.0 + 4.0
                    xf = jnp.minimum(jnp.maximum(xf, 0.0), float(HP - 1))
                    xi = jnp.minimum(xf.astype(jnp.int32), HP - 2)
                    wy_v[qb, pl.ds(off, 16)] = yf - yi.astype(jnp.float32)
                    wx_v[qb, pl.ds(off, 16)] = xf - xi.astype(jnp.float32)
                    par_v[qb, pl.ds(off, 16)] = jnp.bitwise_and(xi, 1)
                    i0 = yi * HPW + lax.shift_right_logical(xi, 1)
                    idx_v[qb, 0, s, pl.ds(j * 16, 16)] = i0
                    idx_v[qb, 1, s, pl.ds(j * 16, 16)] = i0 + 1
                    idx_v[qb, 2, s, pl.ds(j * 16, 16)] = i0 + HPW
                    idx_v[qb, 3, s, pl.ds(j * 16, 16)] = jnp.minimum(
                        i0 + HPW + 1, NTAB - 1)
                    return carry
                lax.fori_loop(0, 8, phase_a, None)

        descs = fire(0, 0)
        for qb in range(NB):
            p = qb % 2
            next_descs = fire(qb + 1, 1 - p) if qb + 1 < NB else []
            for d in descs:
                d.wait()
            descs = next_descs

            def stage(q, carry, p=p):
                base = q * 64
                rts_v[pl.ds(base, 16)] = rt_v[p, q]
                rts_v[pl.ds(base + 16, 16)] = rt_v[p, W + q]
                rts_v[pl.ds(base + 32, 16)] = rt_v[p, 2 * W + q]
                rts_v[pl.ds(base + 48, 16)] = rt_v[p, 3 * W + q]
                return carry
            lax.fori_loop(0, W, stage, None)

            def combine(g, carry, qb=qb):
                off = g * 16
                pav = par_v[qb, pl.ds(off, 16)] * 8
                qa = (off + iota16) * 64 + pav
                for wb in range(NB):
                    wx = wx_v[wb, pl.ds(off, 16)]
                    wy = wy_v[wb, pl.ds(off, 16)]
                    for ci in range(2):
                        ch = 2 * wb + ci
                        a = plsc.load_gather(rts_v, [qa + ch])
                        b_ = plsc.load_gather(rts_v, [qa + (ch + 8)])
                        cc = plsc.load_gather(rts_v, [qa + (ch + 32)])
                        dd = plsc.load_gather(rts_v, [qa + (ch + 40)])
                        top = a + wx * (b_ - a)
                        bot = cc + wx * (dd - cc)
                        out_v[ch, pl.ds(off, 16)] = top + wy * (bot - top)
                return carry
            lax.fori_loop(0, W // 16, combine, None)

            for ch in range(F):
                oi = ch * NB + qb
                pltpu.sync_copy(out_v.at[ch],
                                out_ref.at[oi // F, oi % F, h])

    def row_loop(k, carry):
        do_row(k)
        return carry
    lax.fori_loop(0, ROWS_PW, row_loop, None)


_SC_PARAMS = dict(
    mesh=plsc.VectorSubcoreMesh(core_axis_name="c", subcore_axis_name="s"),
    compiler_params=pltpu.CompilerParams(
        needs_layout_passes=False, use_tc_tiling_on_sc=False),
)


@jax.jit
def kernel(uv, feature_img):
    tab = lax.reshape(feature_img, (NTAB, 2 * F), dimensions=(1, 2, 0))
    run = functools.partial(
        pl.kernel,
        out_type=jax.ShapeDtypeStruct((NB, F, H, W), jnp.float32),
        scratch_types=[
            pltpu.VMEM((NB, W), jnp.float32),
            pltpu.VMEM((NB, W), jnp.float32),
            pltpu.VMEM((NB, W), jnp.float32),
            pltpu.VMEM((NB, W), jnp.float32),
            pltpu.VMEM((NB, W), jnp.int32),
            pltpu.VMEM((NB, 4, 4, 128), jnp.int32),
            pltpu.VMEM((2, 4 * W, 2 * F), jnp.float32),
            pltpu.VMEM((4 * W * 2 * F,), jnp.float32),
            pltpu.VMEM((F, W), jnp.float32),
            pltpu.SemaphoreType.DMA,
            pltpu.SemaphoreType.DMA,
        ],
        **_SC_PARAMS,
    )(_sc_body)
    return run(uv, tab)


# speedup vs baseline: 1.0681x; 1.0264x over previous
import functools

import jax
import jax.numpy as jnp
from jax import lax
from jax.experimental import pallas as pl
from jax.experimental.pallas import tpu as pltpu
from jax.experimental.pallas import tpu_sc as plsc

F = 8
HP = 2056
HPW = HP // 2
NTAB = HP * HPW
NB = 4
W = 512
H = 512
NW = 32
ROWS_PW = H // NW
ROWS_A = (HP + NW - 1) // NW


def _review_body(t3_ref, tab_ref, bufa_v, bufb_v):
    wid = lax.axis_index("s") * 2 + lax.axis_index("c")
    y0 = wid * ROWS_A

    for i in range(ROWS_A):
        y = y0 + i

        @pl.when(y < HP)
        def _(y=y):
            pltpu.sync_copy(t3_ref.at[y], bufa_v)

            def reshuffle(m, carry):
                bufb_v[m, :] = bufa_v[pl.ds(m * 16, 16)]
                return carry
            lax.fori_loop(0, HPW, reshuffle, None)
            pltpu.sync_copy(bufb_v, tab_ref.at[pl.ds(y * HPW, HPW)])


def _sc_body(uv_ref, tab_ref, out_ref, u_v, v_v, wx_v, wy_v, par_v, idx_v,
             rt_v, rts_v, out_v, sem0, sem1):
    wid = lax.axis_index("s") * 2 + lax.axis_index("c")
    iota16 = lax.iota(jnp.int32, 16)
    row0 = wid * ROWS_PW

    def fire(qb, p):
        descs = []
        for cr in range(4):
            for s in range(4):
                descs.append(pltpu.async_copy(
                    tab_ref.at[idx_v.at[qb, cr, s]],
                    rt_v.at[p, pl.ds(cr * W + s * 128, 128)],
                    sem0 if p == 0 else sem1))
        return descs

    def do_row(k):
        h = row0 + k

        for qb in range(NB):
            pltpu.sync_copy(uv_ref.at[qb, 0, h], u_v.at[qb])
            pltpu.sync_copy(uv_ref.at[qb, 1, h], v_v.at[qb])
            for s in range(4):
                def phase_a(j, carry, qb=qb, s=s):
                    off = (s * 8 + j) * 16
                    yf = u_v[qb, pl.ds(off, 16)] * 2048.0 + 4.0
                    yf = jnp.minimum(jnp.maximum(yf, 0.0), float(HP - 1))
                    yi = jnp.minimum(yf.astype(jnp.int32), HP - 2)
                    xf = v_v[qb, pl.ds(off, 16)] * 2048.0 + 4.0
                    xf = jnp.minimum(jnp.maximum(xf, 0.0), float(HP - 1))
                    xi = jnp.minimum(xf.astype(jnp.int32), HP - 2)
                    wy_v[qb, pl.ds(off, 16)] = yf - yi.astype(jnp.float32)
                    wx_v[qb, pl.ds(off, 16)] = xf - xi.astype(jnp.float32)
                    par_v[qb, pl.ds(off, 16)] = jnp.bitwise_and(xi, 1)
                    i0 = yi * HPW + lax.shift_right_logical(xi, 1)
                    idx_v[qb, 0, s, pl.ds(j * 16, 16)] = i0
                    idx_v[qb, 1, s, pl.ds(j * 16, 16)] = i0 + 1
                    idx_v[qb, 2, s, pl.ds(j * 16, 16)] = i0 + HPW
                    idx_v[qb, 3, s, pl.ds(j * 16, 16)] = jnp.minimum(
                        i0 + HPW + 1, NTAB - 1)
                    return carry
                lax.fori_loop(0, 8, phase_a, None)

        descs = fire(0, 0)
        for qb in range(NB):
            p = qb % 2
            next_descs = fire(qb + 1, 1 - p) if qb + 1 < NB else []
            for d in descs:
                d.wait()
            descs = next_descs

            def stage(q, carry, p=p):
                base = q * 64
                rts_v[pl.ds(base, 16)] = rt_v[p, q]
                rts_v[pl.ds(base + 16, 16)] = rt_v[p, W + q]
                rts_v[pl.ds(base + 32, 16)] = rt_v[p, 2 * W + q]
                rts_v[pl.ds(base + 48, 16)] = rt_v[p, 3 * W + q]
                return carry
            lax.fori_loop(0, W, stage, None)

            def combine(g, carry, qb=qb):
                off = g * 16
                pav = par_v[qb, pl.ds(off, 16)] * 8
                qa = (off + iota16) * 64 + pav
                for wb in range(NB):
                    wx = wx_v[wb, pl.ds(off, 16)]
                    wy = wy_v[wb, pl.ds(off, 16)]
                    for ci in range(2):
                        ch = 2 * wb + ci
                        a = plsc.load_gather(rts_v, [qa + ch])
                        b_ = plsc.load_gather(rts_v, [qa + (ch + 8)])
                        cc = plsc.load_gather(rts_v, [qa + (ch + 32)])
                        dd = plsc.load_gather(rts_v, [qa + (ch + 40)])
                        top = a + wx * (b_ - a)
                        bot = cc + wx * (dd - cc)
                        out_v[ch, pl.ds(off, 16)] = top + wy * (bot - top)
                return carry
            lax.fori_loop(0, W // 16, combine, None)

            for ch in range(F):
                oi = ch * NB + qb
                pltpu.sync_copy(out_v.at[ch],
                                out_ref.at[oi // F, oi % F, h])

    def row_loop(k, carry):
        do_row(k)
        return carry
    lax.fori_loop(0, ROWS_PW, row_loop, None)


_SC_PARAMS = dict(
    mesh=plsc.VectorSubcoreMesh(core_axis_name="c", subcore_axis_name="s"),
    compiler_params=pltpu.CompilerParams(
        needs_layout_passes=False, use_tc_tiling_on_sc=False),
)


@jax.jit
def kernel(uv, feature_img):
    t3 = jnp.transpose(feature_img, (1, 2, 0)).reshape(HP, HP * F)
    review = functools.partial(
        pl.kernel,
        out_type=jax.ShapeDtypeStruct((NTAB, 2 * F), jnp.float32),
        scratch_types=[
            pltpu.VMEM((HP * F,), jnp.float32),
            pltpu.VMEM((HPW, 2 * F), jnp.float32),
        ],
        **_SC_PARAMS,
    )(_review_body)
    tab = review(t3)
    run = functools.partial(
        pl.kernel,
        out_type=jax.ShapeDtypeStruct((NB, F, H, W), jnp.float32),
        scratch_types=[
            pltpu.VMEM((NB, W), jnp.float32),
            pltpu.VMEM((NB, W), jnp.float32),
            pltpu.VMEM((NB, W), jnp.float32),
            pltpu.VMEM((NB, W), jnp.float32),
            pltpu.VMEM((NB, W), jnp.int32),
            pltpu.VMEM((NB, 4, 4, 128), jnp.int32),
            pltpu.VMEM((2, 4 * W, 2 * F), jnp.float32),
            pltpu.VMEM((4 * W * 2 * F,), jnp.float32),
            pltpu.VMEM((F, W), jnp.float32),
            pltpu.SemaphoreType.DMA,
            pltpu.SemaphoreType.DMA,
        ],
        **_SC_PARAMS,
    )(_sc_body)
    return run(uv, tab)


# speedup vs baseline: 1.0997x; 1.0296x over previous
import functools

import jax
import jax.numpy as jnp
from jax import lax
from jax.experimental import pallas as pl
from jax.experimental.pallas import tpu as pltpu
from jax.experimental.pallas import tpu_sc as plsc

F = 8
HP = 2056
HPW = HP // 2
NTAB = HP * HPW
NB = 4
W = 512
H = 512
NW = 32
ROWS_PW = H // NW
ROWS_A = (HP + NW - 1) // NW


def _review_body(t3_ref, tab_ref, bufa_v, bufb_v):
    wid = lax.axis_index("s") * 2 + lax.axis_index("c")
    y0 = wid * ROWS_A

    for i in range(ROWS_A):
        y = y0 + i

        @pl.when(y < HP)
        def _(y=y):
            pltpu.sync_copy(t3_ref.at[y], bufa_v)

            def reshuffle(m, carry):
                bufb_v[m, :] = bufa_v[pl.ds(m * 16, 16)]
                return carry
            lax.fori_loop(0, HPW, reshuffle, None)
            pltpu.sync_copy(bufb_v, tab_ref.at[pl.ds(y * HPW, HPW)])


def _sc_body(uv_ref, tab_ref, out_ref, u_v, v_v, wx_v, wy_v, par_v, idx_v,
             rt_v, rts_v, out_v, sem0, sem1, usem, osem0, osem1):
    wid = lax.axis_index("s") * 2 + lax.axis_index("c")
    iota16 = lax.iota(jnp.int32, 16)
    row0 = wid * ROWS_PW

    def fire(qb, p):
        descs = []
        for cr in range(4):
            for s in range(4):
                descs.append(pltpu.async_copy(
                    tab_ref.at[idx_v.at[qb, cr, s]],
                    rt_v.at[p, pl.ds(cr * W + s * 128, 128)],
                    sem0 if p == 0 else sem1))
        return descs

    def do_row(k):
        h = row0 + k

        uvds = []
        for qb in range(NB):
            uvds.append(pltpu.async_copy(uv_ref.at[qb, 0, h], u_v.at[qb], usem))
            uvds.append(pltpu.async_copy(uv_ref.at[qb, 1, h], v_v.at[qb], usem))
        for d in uvds:
            d.wait()
        for qb in range(NB):
            for s in range(4):
                def phase_a(j, carry, qb=qb, s=s):
                    off = (s * 8 + j) * 16
                    yf = u_v[qb, pl.ds(off, 16)] * 2048.0 + 4.0
                    yf = jnp.minimum(jnp.maximum(yf, 0.0), float(HP - 1))
                    yi = jnp.minimum(yf.astype(jnp.int32), HP - 2)
                    xf = v_v[qb, pl.ds(off, 16)] * 2048.0 + 4.0
                    xf = jnp.minimum(jnp.maximum(xf, 0.0), float(HP - 1))
                    xi = jnp.minimum(xf.astype(jnp.int32), HP - 2)
                    wy_v[qb, pl.ds(off, 16)] = yf - yi.astype(jnp.float32)
                    wx_v[qb, pl.ds(off, 16)] = xf - xi.astype(jnp.float32)
                    par_v[qb, pl.ds(off, 16)] = jnp.bitwise_and(xi, 1)
                    i0 = yi * HPW + lax.shift_right_logical(xi, 1)
                    idx_v[qb, 0, s, pl.ds(j * 16, 16)] = i0
                    idx_v[qb, 1, s, pl.ds(j * 16, 16)] = i0 + 1
                    idx_v[qb, 2, s, pl.ds(j * 16, 16)] = i0 + HPW
                    idx_v[qb, 3, s, pl.ds(j * 16, 16)] = jnp.minimum(
                        i0 + HPW + 1, NTAB - 1)
                    return carry
                lax.fori_loop(0, 8, phase_a, None)

        descs = fire(0, 0)
        for qb in range(NB):
            p = qb % 2
            next_descs = fire(qb + 1, 1 - p) if qb + 1 < NB else []
            for d in descs:
                d.wait()
            descs = next_descs

            def stage(q, carry, p=p):
                base = q * 64
                rts_v[pl.ds(base, 16)] = rt_v[p, q]
                rts_v[pl.ds(base + 16, 16)] = rt_v[p, W + q]
                rts_v[pl.ds(base + 32, 16)] = rt_v[p, 2 * W + q]
                rts_v[pl.ds(base + 48, 16)] = rt_v[p, 3 * W + q]
                return carry
            lax.fori_loop(0, W, stage, None)

            po = qb % 2
            posem = osem0 if po == 0 else osem1

            @pl.when(k * NB + qb >= 2)
            def _(po=po, posem=posem):
                for ch in range(F):
                    pltpu.make_async_copy(
                        out_v.at[po, ch], out_ref.at[0, 0, 0], posem).wait()

            def combine(g, carry, qb=qb, po=po):
                off = g * 16
                pav = par_v[qb, pl.ds(off, 16)] * 8
                qa = (off + iota16) * 64 + pav
                for wb in range(NB):
                    wx = wx_v[wb, pl.ds(off, 16)]
                    wy = wy_v[wb, pl.ds(off, 16)]
                    for ci in range(2):
                        ch = 2 * wb + ci
                        a = plsc.load_gather(rts_v, [qa + ch])
                        b_ = plsc.load_gather(rts_v, [qa + (ch + 8)])
                        cc = plsc.load_gather(rts_v, [qa + (ch + 32)])
                        dd = plsc.load_gather(rts_v, [qa + (ch + 40)])
                        top = a + wx * (b_ - a)
                        bot = cc + wx * (dd - cc)
                        out_v[po, ch, pl.ds(off, 16)] = top + wy * (bot - top)
                return carry
            lax.fori_loop(0, W // 16, combine, None)

            for ch in range(F):
                oi = ch * NB + qb
                pltpu.async_copy(out_v.at[po, ch],
                                 out_ref.at[oi // F, oi % F, h], posem)

    def row_loop(k, carry):
        do_row(k)
        return carry
    lax.fori_loop(0, ROWS_PW, row_loop, None)
    for psem in (osem0, osem1):
        for ch in range(F):
            pltpu.make_async_copy(
                out_v.at[0, ch], out_ref.at[0, 0, 0], psem).wait()


_SC_PARAMS = dict(
    mesh=plsc.VectorSubcoreMesh(core_axis_name="c", subcore_axis_name="s"),
    compiler_params=pltpu.CompilerParams(
        needs_layout_passes=False, use_tc_tiling_on_sc=False),
)


@jax.jit
def kernel(uv, feature_img):
    t3 = jnp.transpose(feature_img, (1, 2, 0)).reshape(HP, HP * F)
    review = functools.partial(
        pl.kernel,
        out_type=jax.ShapeDtypeStruct((NTAB, 2 * F), jnp.float32),
        scratch_types=[
            pltpu.VMEM((HP * F,), jnp.float32),
            pltpu.VMEM((HPW, 2 * F), jnp.float32),
        ],
        **_SC_PARAMS,
    )(_review_body)
    tab = review(t3)
    run = functools.partial(
        pl.kernel,
        out_type=jax.ShapeDtypeStruct((NB, F, H, W), jnp.float32),
        scratch_types=[
            pltpu.VMEM((NB, W), jnp.float32),
            pltpu.VMEM((NB, W), jnp.float32),
            pltpu.VMEM((NB, W), jnp.float32),
            pltpu.VMEM((NB, W), jnp.float32),
            pltpu.VMEM((NB, W), jnp.int32),
            pltpu.VMEM((NB, 4, 4, 128), jnp.int32),
            pltpu.VMEM((2, 4 * W, 2 * F), jnp.float32),
            pltpu.VMEM((4 * W * 2 * F,), jnp.float32),
            pltpu.VMEM((2, F, W), jnp.float32),
            pltpu.SemaphoreType.DMA,
            pltpu.SemaphoreType.DMA,
            pltpu.SemaphoreType.DMA,
            pltpu.SemaphoreType.DMA,
            pltpu.SemaphoreType.DMA,
        ],
        **_SC_PARAMS,
    )(_sc_body)
    return run(uv, tab)


# speedup vs baseline: 1.8309x; 1.6649x over previous
import functools

import jax
import jax.numpy as jnp
from jax import lax
from jax.experimental import pallas as pl
from jax.experimental.pallas import tpu as pltpu
from jax.experimental.pallas import tpu_sc as plsc

F = 8
HP = 2056
HPW = HP // 2
NTAB = HP * HPW
NB = 4
W = 512
H = 512
NW = 32
ROWS_PW = H // NW
ROWS_A = (HP + NW - 1) // NW


def _review_body(f_ref, tab_ref, src_v, bufb_v, lsem):
    wid = lax.axis_index("s") * 2 + lax.axis_index("c")
    iota16 = lax.iota(jnp.int32, 16)
    y0 = wid * ROWS_A
    offs = (jnp.bitwise_and(iota16, 7) * HP
            + lax.shift_right_logical(iota16, 3))

    for i in range(ROWS_A):
        y = y0 + i

        @pl.when(y < HP)
        def _(y=y):
            descs = []
            for c in range(F):
                descs.append(pltpu.async_copy(
                    f_ref.at[c, y], src_v.at[pl.ds(c * HP, HP)], lsem))
            for d in descs:
                d.wait()

            def reshuffle(xp, carry):
                bufb_v[xp, :] = plsc.load_gather(src_v, [offs + 2 * xp])
                return carry
            lax.fori_loop(0, HPW, reshuffle, None)
            pltpu.sync_copy(bufb_v, tab_ref.at[pl.ds(y * HPW, HPW)])


def _sc_body(uv_ref, tab_ref, out_ref, u_v, v_v, wx_v, wy_v, par_v, idx_v,
             rt_v, rts_v, out_v, sem0, sem1, usem, osem0, osem1):
    wid = lax.axis_index("s") * 2 + lax.axis_index("c")
    iota16 = lax.iota(jnp.int32, 16)
    row0 = wid * ROWS_PW

    def fire(qb, p):
        descs = []
        for cr in range(4):
            for s in range(4):
                descs.append(pltpu.async_copy(
                    tab_ref.at[idx_v.at[qb, cr, s]],
                    rt_v.at[p, pl.ds(cr * W + s * 128, 128)],
                    sem0 if p == 0 else sem1))
        return descs

    def do_row(k):
        h = row0 + k

        uvds = []
        for qb in range(NB):
            uvds.append(pltpu.async_copy(uv_ref.at[qb, 0, h], u_v.at[qb], usem))
            uvds.append(pltpu.async_copy(uv_ref.at[qb, 1, h], v_v.at[qb], usem))
        for d in uvds:
            d.wait()
        for qb in range(NB):
            for s in range(4):
                def phase_a(j, carry, qb=qb, s=s):
                    off = (s * 8 + j) * 16
                    yf = u_v[qb, pl.ds(off, 16)] * 2048.0 + 4.0
                    yf = jnp.minimum(jnp.maximum(yf, 0.0), float(HP - 1))
                    yi = jnp.minimum(yf.astype(jnp.int32), HP - 2)
                    xf = v_v[qb, pl.ds(off, 16)] * 2048.0 + 4.0
                    xf = jnp.minimum(jnp.maximum(xf, 0.0), float(HP - 1))
                    xi = jnp.minimum(xf.astype(jnp.int32), HP - 2)
                    wy_v[qb, pl.ds(off, 16)] = yf - yi.astype(jnp.float32)
                    wx_v[qb, pl.ds(off, 16)] = xf - xi.astype(jnp.float32)
                    par_v[qb, pl.ds(off, 16)] = jnp.bitwise_and(xi, 1)
                    i0 = yi * HPW + lax.shift_right_logical(xi, 1)
                    idx_v[qb, 0, s, pl.ds(j * 16, 16)] = i0
                    idx_v[qb, 1, s, pl.ds(j * 16, 16)] = i0 + 1
                    idx_v[qb, 2, s, pl.ds(j * 16, 16)] = i0 + HPW
                    idx_v[qb, 3, s, pl.ds(j * 16, 16)] = jnp.minimum(
                        i0 + HPW + 1, NTAB - 1)
                    return carry
                lax.fori_loop(0, 8, phase_a, None)

        descs = fire(0, 0)
        for qb in range(NB):
            p = qb % 2
            next_descs = fire(qb + 1, 1 - p) if qb + 1 < NB else []
            for d in descs:
                d.wait()
            descs = next_descs

            def stage(q, carry, p=p):
                base = q * 64
                rts_v[pl.ds(base, 16)] = rt_v[p, q]
                rts_v[pl.ds(base + 16, 16)] = rt_v[p, W + q]
                rts_v[pl.ds(base + 32, 16)] = rt_v[p, 2 * W + q]
                rts_v[pl.ds(base + 48, 16)] = rt_v[p, 3 * W + q]
                return carry
            lax.fori_loop(0, W, stage, None)

            po = qb % 2
            posem = osem0 if po == 0 else osem1

            @pl.when(k * NB + qb >= 2)
            def _(po=po, posem=posem):
                for ch in range(F):
                    pltpu.make_async_copy(
                        out_v.at[po, ch], out_ref.at[0, 0, 0], posem).wait()

            def combine(g, carry, qb=qb, po=po):
                off = g * 16
                pav = par_v[qb, pl.ds(off, 16)] * 8
                qa = (off + iota16) * 64 + pav
                for wb in range(NB):
                    wx = wx_v[wb, pl.ds(off, 16)]
                    wy = wy_v[wb, pl.ds(off, 16)]
                    for ci in range(2):
                        ch = 2 * wb + ci
                        a = plsc.load_gather(rts_v, [qa + ch])
                        b_ = plsc.load_gather(rts_v, [qa + (ch + 8)])
                        cc = plsc.load_gather(rts_v, [qa + (ch + 32)])
                        dd = plsc.load_gather(rts_v, [qa + (ch + 40)])
                        top = a + wx * (b_ - a)
                        bot = cc + wx * (dd - cc)
                        out_v[po, ch, pl.ds(off, 16)] = top + wy * (bot - top)
                return carry
            lax.fori_loop(0, W // 16, combine, None)

            for ch in range(F):
                oi = ch * NB + qb
                pltpu.async_copy(out_v.at[po, ch],
                                 out_ref.at[oi // F, oi % F, h], posem)

    def row_loop(k, carry):
        do_row(k)
        return carry
    lax.fori_loop(0, ROWS_PW, row_loop, None)
    for psem in (osem0, osem1):
        for ch in range(F):
            pltpu.make_async_copy(
                out_v.at[0, ch], out_ref.at[0, 0, 0], psem).wait()


_SC_PARAMS = dict(
    mesh=plsc.VectorSubcoreMesh(core_axis_name="c", subcore_axis_name="s"),
    compiler_params=pltpu.CompilerParams(
        needs_layout_passes=False, use_tc_tiling_on_sc=False),
)


@jax.jit
def kernel(uv, feature_img):
    review = functools.partial(
        pl.kernel,
        out_type=jax.ShapeDtypeStruct((NTAB, 2 * F), jnp.float32),
        scratch_types=[
            pltpu.VMEM((F * HP,), jnp.float32),
            pltpu.VMEM((HPW, 2 * F), jnp.float32),
            pltpu.SemaphoreType.DMA,
        ],
        **_SC_PARAMS,
    )(_review_body)
    tab = review(feature_img)
    run = functools.partial(
        pl.kernel,
        out_type=jax.ShapeDtypeStruct((NB, F, H, W), jnp.float32),
        scratch_types=[
            pltpu.VMEM((NB, W), jnp.float32),
            pltpu.VMEM((NB, W), jnp.float32),
            pltpu.VMEM((NB, W), jnp.float32),
            pltpu.VMEM((NB, W), jnp.float32),
            pltpu.VMEM((NB, W), jnp.int32),
            pltpu.VMEM((NB, 4, 4, 128), jnp.int32),
            pltpu.VMEM((2, 4 * W, 2 * F), jnp.float32),
            pltpu.VMEM((4 * W * 2 * F,), jnp.float32),
            pltpu.VMEM((2, F, W), jnp.float32),
            pltpu.SemaphoreType.DMA,
            pltpu.SemaphoreType.DMA,
            pltpu.SemaphoreType.DMA,
            pltpu.SemaphoreType.DMA,
            pltpu.SemaphoreType.DMA,
        ],
        **_SC_PARAMS,
    )(_sc_body)
    return run(uv, tab)
